# Initial kernel scaffold; baseline (speedup 1.0000x reference)
import functools

import jax
import jax.numpy as jnp
from jax import lax
from jax.experimental import pallas as pl
from jax.experimental.pallas import tpu as pltpu
from jax.experimental.pallas import tpu_sc as plsc

NC = 2
NS = 16
CHUNK = 128


def _make_deg_kernel(N_pad, E_pad):
    per_w = E_pad // (NC * NS)
    n_chunks = per_w // CHUNK
    rpt = N_pad // NS
    mesh = plsc.VectorSubcoreMesh(core_axis_name="c", subcore_axis_name="s")

    @functools.partial(
        pl.kernel,
        out_type=jax.ShapeDtypeStruct((NC, N_pad, 16), jnp.float32),
        mesh=mesh,
        scratch_types=[
            pltpu.VMEM((CHUNK,), jnp.int32),
            pltpu.VMEM((CHUNK, 16), jnp.float32),
            pltpu.VMEM((rpt, 16), jnp.float32),
            pltpu.VMEM_SHARED((N_pad, 16), jnp.float32),
        ],
    )
    def deg_k(dst_hbm, ones_hbm, zeros_hbm, out_hbm, idx_v, ones_v, buf_v, acc_sh):
        c = lax.axis_index("c")
        s = lax.axis_index("s")
        w = c * NS + s
        base = w * per_w
        pltpu.sync_copy(ones_hbm, ones_v)
        pltpu.sync_copy(zeros_hbm.at[pl.ds(s * rpt, rpt)],
                        acc_sh.at[pl.ds(s * rpt, rpt)])
        plsc.subcore_barrier()

        def body(i, carry):
            off = base + i * CHUNK
            pltpu.sync_copy(dst_hbm.at[pl.ds(off, CHUNK)], idx_v)
            pltpu.sync_copy(ones_v, acc_sh.at[idx_v], add=True)
            return carry

        lax.fori_loop(0, n_chunks, body, 0)
        plsc.subcore_barrier()
        pltpu.sync_copy(acc_sh.at[pl.ds(s * rpt, rpt)], buf_v)
        pltpu.sync_copy(buf_v, out_hbm.at[c].at[pl.ds(s * rpt, rpt)])

    return deg_k


def _make_layer_kernel(N, N_pad, D, E_pad):
    per_w = E_pad // (NC * NS)
    n_chunks = per_w // CHUNK
    rpt = N_pad // NS
    mesh = plsc.VectorSubcoreMesh(core_axis_name="c", subcore_axis_name="s")

    @functools.partial(
        pl.kernel,
        out_type=jax.ShapeDtypeStruct((NC, N_pad, D), jnp.float32),
        mesh=mesh,
        scratch_types=[
            pltpu.VMEM((CHUNK,), jnp.int32),
            pltpu.VMEM((CHUNK,), jnp.int32),
            pltpu.VMEM((CHUNK, D), jnp.float32),
            pltpu.VMEM_SHARED((N_pad, D), jnp.float32),
            pltpu.SemaphoreType.DMA,
        ],
    )
    def layer_k(src_hbm, dst_hbm, g_hbm, zeros_hbm, out_hbm,
                idx_s, idx_d, rows_v, acc_sh, sem):
        c = lax.axis_index("c")
        s = lax.axis_index("s")
        w = c * NS + s
        base = w * per_w
        pltpu.sync_copy(zeros_hbm.at[pl.ds(s * rpt, rpt)],
                        acc_sh.at[pl.ds(s * rpt, rpt)])
        plsc.subcore_barrier()

        def body(i, carry):
            off = base + i * CHUNK
            pltpu.sync_copy(src_hbm.at[pl.ds(off, CHUNK)], idx_s)
            pltpu.sync_copy(dst_hbm.at[pl.ds(off, CHUNK)], idx_d)
            pltpu.async_copy(g_hbm.at[idx_s], rows_v, sem).wait()
            pltpu.sync_copy(rows_v, acc_sh.at[idx_d], add=True)
            return carry

        lax.fori_loop(0, n_chunks, body, 0)
        plsc.subcore_barrier()
        n_full, rem = divmod(rpt, CHUNK)
        for j in range(n_full):
            r0 = s * rpt + j * CHUNK
            pltpu.sync_copy(acc_sh.at[pl.ds(r0, CHUNK)], rows_v)
            pltpu.sync_copy(rows_v, out_hbm.at[c].at[pl.ds(r0, CHUNK)])
        if rem:
            r0 = s * rpt + n_full * CHUNK
            pltpu.sync_copy(acc_sh.at[pl.ds(r0, rem)], rows_v.at[pl.ds(0, rem)])
            pltpu.sync_copy(rows_v.at[pl.ds(0, rem)], out_hbm.at[c].at[pl.ds(r0, rem)])

    return layer_k


def _prep1_tc(x, W1, degp2):
    N, D_in = x.shape
    D_h = W1.shape[1]

    def body(x_ref, w_ref, degp_ref, g_ref, dinv_ref):
        dp = degp_ref[...]
        deg = dp[:, 0:1] + dp[:, 1:2] + 1.0
        dinv = lax.rsqrt(deg)
        h = jnp.dot(x_ref[...], w_ref[...], preferred_element_type=jnp.float32)
        g_ref[...] = h * dinv
        dinv_ref[...] = dinv

    return pl.pallas_call(
        body,
        out_shape=[
            jax.ShapeDtypeStruct((N, D_h), jnp.float32),
            jax.ShapeDtypeStruct((N, 1), jnp.float32),
        ],
    )(x, W1, degp2)


def _mid_tc(p, g1, dinv, b1, W2):
    N, D = g1.shape
    D_out = W2.shape[1]

    def body(p_ref, g1_ref, dinv_ref, b1_ref, w2_ref, g2_ref):
        sall = p_ref[0] + p_ref[1] + g1_ref[...]
        z = jnp.maximum(sall * dinv_ref[...] + b1_ref[...], 0.0)
        h2 = jnp.dot(z, w2_ref[...], preferred_element_type=jnp.float32)
        g2_ref[...] = h2 * dinv_ref[...]

    return pl.pallas_call(
        body,
        out_shape=jax.ShapeDtypeStruct((N, D_out), jnp.float32),
    )(p, g1, dinv, b1, W2)


def _final_tc(q, g2, dinv, b2):
    N, D = g2.shape

    def body(q_ref, g2_ref, dinv_ref, b2_ref, o_ref):
        sall = q_ref[0] + q_ref[1] + g2_ref[...]
        o_ref[...] = sall * dinv_ref[...] + b2_ref[...]

    return pl.pallas_call(
        body,
        out_shape=jax.ShapeDtypeStruct((N, D), jnp.float32),
    )(q, g2, dinv, b2)


def kernel(x, edge_index, W1, b1, W2, b2):
    N, D_in = x.shape
    E = edge_index.shape[1]
    D = W1.shape[1]

    W_TOT = NC * NS
    E_pad = ((E + W_TOT * CHUNK - 1) // (W_TOT * CHUNK)) * (W_TOT * CHUNK)
    pad = E_pad - E
    src = edge_index[0].astype(jnp.int32)
    dst = edge_index[1].astype(jnp.int32)
    if pad:
        src = jnp.concatenate([src, jnp.zeros((pad,), jnp.int32)])
        dst = jnp.concatenate([dst, N + (jnp.arange(pad, dtype=jnp.int32) % NS)])
    N_pad = ((N + NS) // NS) * NS

    ones16 = jnp.zeros((CHUNK, 16), jnp.float32).at[:, 0].set(1.0)
    zeros16 = jnp.zeros((N_pad, 16), jnp.float32)
    zerosD = jnp.zeros((N_pad, D), jnp.float32)

    deg_k = _make_deg_kernel(N_pad, E_pad)
    layer_k = _make_layer_kernel(N, N_pad, D, E_pad)

    degp = deg_k(dst, ones16, zeros16)
    degp2 = jnp.transpose(degp[:, :N, 0])

    g1, dinv = _prep1_tc(x, W1, degp2)
    p = layer_k(src, dst, g1, zerosD)
    g2 = _mid_tc(p[:, :N, :], g1, dinv, jnp.reshape(b1, (1, D)), W2)
    q = layer_k(src, dst, g2, zerosD)
    out = _final_tc(q[:, :N, :], g2, dinv, jnp.reshape(b2, (1, W2.shape[1])))
    return out


# speedup vs baseline: 10.3454x; 10.3454x over previous
import functools

import jax
import jax.numpy as jnp
from jax import lax
from jax.experimental import pallas as pl
from jax.experimental.pallas import tpu as pltpu
from jax.experimental.pallas import tpu_sc as plsc

NC = 2
NS = 16
CHUNK = 128


def _make_deg_kernel(N_pad, E_pad):
    per_w = E_pad // (NC * NS)
    n_chunks = per_w // CHUNK
    rpt = N_pad // NS
    mesh = plsc.VectorSubcoreMesh(core_axis_name="c", subcore_axis_name="s")

    @functools.partial(
        pl.kernel,
        out_type=jax.ShapeDtypeStruct((NC, N_pad), jnp.float32),
        mesh=mesh,
        scratch_types=[
            pltpu.VMEM((CHUNK,), jnp.int32),
            pltpu.VMEM((CHUNK,), jnp.float32),
            pltpu.VMEM((rpt,), jnp.float32),
            pltpu.VMEM_SHARED((N_pad,), jnp.float32),
        ],
    )
    def deg_k(dst_hbm, ones_hbm, zeros_hbm, out_hbm, idx_v, ones_v, buf_v, acc_sh):
        c = lax.axis_index("c")
        s = lax.axis_index("s")
        w = c * NS + s
        base = w * per_w
        pltpu.sync_copy(ones_hbm, ones_v)
        pltpu.sync_copy(zeros_hbm.at[pl.ds(0, rpt)], buf_v)
        pltpu.sync_copy(buf_v, acc_sh.at[pl.ds(s * rpt, rpt)])
        plsc.subcore_barrier()

        def body(i, carry):
            off = base + i * CHUNK
            pltpu.sync_copy(dst_hbm.at[pl.ds(off, CHUNK)], idx_v)
            pltpu.sync_copy(ones_v, acc_sh.at[idx_v], add=True)
            return carry

        lax.fori_loop(0, n_chunks, body, 0)
        plsc.subcore_barrier()
        pltpu.sync_copy(acc_sh.at[pl.ds(s * rpt, rpt)], buf_v)
        pltpu.sync_copy(buf_v, out_hbm.at[c].at[pl.ds(s * rpt, rpt)])

    return deg_k


def _make_layer_kernel(N, N_pad, D, E_pad):
    per_w = E_pad // (NC * NS)
    n_chunks = per_w // CHUNK
    rpt = N_pad // NS
    mesh = plsc.VectorSubcoreMesh(core_axis_name="c", subcore_axis_name="s")

    @functools.partial(
        pl.kernel,
        out_type=jax.ShapeDtypeStruct((NC, N_pad, D), jnp.float32),
        mesh=mesh,
        scratch_types=[
            pltpu.VMEM((CHUNK,), jnp.int32),
            pltpu.VMEM((CHUNK,), jnp.int32),
            pltpu.VMEM((CHUNK, D), jnp.float32),
            pltpu.VMEM_SHARED((N_pad, D), jnp.float32),
            pltpu.SemaphoreType.DMA,
        ],
    )
    def layer_k(src_hbm, dst_hbm, g_hbm, zeros_hbm, out_hbm,
                idx_s, idx_d, rows_v, acc_sh, sem):
        c = lax.axis_index("c")
        s = lax.axis_index("s")
        w = c * NS + s
        base = w * per_w
        pltpu.sync_copy(zeros_hbm.at[pl.ds(0, CHUNK)], rows_v)
        n_full0, rem0 = divmod(rpt, CHUNK)
        for j in range(n_full0):
            pltpu.sync_copy(rows_v, acc_sh.at[pl.ds(s * rpt + j * CHUNK, CHUNK)])
        if rem0:
            pltpu.sync_copy(rows_v.at[pl.ds(0, rem0)],
                            acc_sh.at[pl.ds(s * rpt + n_full0 * CHUNK, rem0)])
        plsc.subcore_barrier()

        def body(i, carry):
            off = base + i * CHUNK
            pltpu.sync_copy(src_hbm.at[pl.ds(off, CHUNK)], idx_s)
            pltpu.sync_copy(dst_hbm.at[pl.ds(off, CHUNK)], idx_d)
            pltpu.async_copy(g_hbm.at[idx_s], rows_v, sem).wait()
            pltpu.sync_copy(rows_v, acc_sh.at[idx_d], add=True)
            return carry

        lax.fori_loop(0, n_chunks, body, 0)
        plsc.subcore_barrier()
        n_full, rem = divmod(rpt, CHUNK)
        for j in range(n_full):
            r0 = s * rpt + j * CHUNK
            pltpu.sync_copy(acc_sh.at[pl.ds(r0, CHUNK)], rows_v)
            pltpu.sync_copy(rows_v, out_hbm.at[c].at[pl.ds(r0, CHUNK)])
        if rem:
            r0 = s * rpt + n_full * CHUNK
            pltpu.sync_copy(acc_sh.at[pl.ds(r0, rem)], rows_v.at[pl.ds(0, rem)])
            pltpu.sync_copy(rows_v.at[pl.ds(0, rem)], out_hbm.at[c].at[pl.ds(r0, rem)])

    return layer_k


def _prep1_tc(x, W1, degp2):
    N, D_in = x.shape
    D_h = W1.shape[1]

    def body(x_ref, w_ref, degp_ref, g_ref, dinv_ref):
        dp = degp_ref[...]
        deg = dp[:, 0:1] + dp[:, 1:2] + 1.0
        dinv = lax.rsqrt(deg)
        h = jnp.dot(x_ref[...], w_ref[...], preferred_element_type=jnp.float32)
        g_ref[...] = h * dinv
        dinv_ref[...] = dinv

    return pl.pallas_call(
        body,
        out_shape=[
            jax.ShapeDtypeStruct((N, D_h), jnp.float32),
            jax.ShapeDtypeStruct((N, 1), jnp.float32),
        ],
    )(x, W1, degp2)


def _mid_tc(p, g1, dinv, b1, W2):
    N, D = g1.shape
    D_out = W2.shape[1]

    def body(p_ref, g1_ref, dinv_ref, b1_ref, w2_ref, g2_ref):
        sall = p_ref[0] + p_ref[1] + g1_ref[...]
        z = jnp.maximum(sall * dinv_ref[...] + b1_ref[...], 0.0)
        h2 = jnp.dot(z, w2_ref[...], preferred_element_type=jnp.float32)
        g2_ref[...] = h2 * dinv_ref[...]

    return pl.pallas_call(
        body,
        out_shape=jax.ShapeDtypeStruct((N, D_out), jnp.float32),
    )(p, g1, dinv, b1, W2)


def _final_tc(q, g2, dinv, b2):
    N, D = g2.shape

    def body(q_ref, g2_ref, dinv_ref, b2_ref, o_ref):
        sall = q_ref[0] + q_ref[1] + g2_ref[...]
        o_ref[...] = sall * dinv_ref[...] + b2_ref[...]

    return pl.pallas_call(
        body,
        out_shape=jax.ShapeDtypeStruct((N, D), jnp.float32),
    )(q, g2, dinv, b2)


def kernel(x, edge_index, W1, b1, W2, b2):
    N, D_in = x.shape
    E = edge_index.shape[1]
    D = W1.shape[1]

    W_TOT = NC * NS
    E_pad = ((E + W_TOT * CHUNK - 1) // (W_TOT * CHUNK)) * (W_TOT * CHUNK)
    pad = E_pad - E
    src = edge_index[0].astype(jnp.int32)
    dst = edge_index[1].astype(jnp.int32)
    if pad:
        src = jnp.concatenate([src, jnp.zeros((pad,), jnp.int32)])
        dst = jnp.concatenate([dst, N + (jnp.arange(pad, dtype=jnp.int32) % NS)])
    N_pad = ((N + NS * 128) // (NS * 128)) * (NS * 128)

    ones1 = jnp.ones((CHUNK,), jnp.float32)
    zeros1 = jnp.zeros((N_pad,), jnp.float32)
    zerosD = jnp.zeros((N_pad, D), jnp.float32)

    deg_k = _make_deg_kernel(N_pad, E_pad)
    layer_k = _make_layer_kernel(N, N_pad, D, E_pad)

    degp = deg_k(dst, ones1, zeros1)
    degp2 = jnp.transpose(degp[:, :N])

    g1, dinv = _prep1_tc(x, W1, degp2)
    p = layer_k(src, dst, g1, zerosD)
    g2 = _mid_tc(p[:, :N, :], g1, dinv, jnp.reshape(b1, (1, D)), W2)
    q = layer_k(src, dst, g2, zerosD)
    out = _final_tc(q[:, :N, :], g2, dinv, jnp.reshape(b2, (1, W2.shape[1])))
    return out


# speedup vs baseline: 16.0182x; 1.5483x over previous
import functools

import jax
import jax.numpy as jnp
from jax import lax
from jax.experimental import pallas as pl
from jax.experimental.pallas import tpu as pltpu
from jax.experimental.pallas import tpu_sc as plsc

NC = 2
NS = 16
CHUNK = 128


def _make_deg_kernel(N_pad, E_pad):
    per_w = E_pad // (NC * NS)
    n_chunks = per_w // CHUNK
    rpt = N_pad // NS
    mesh = plsc.VectorSubcoreMesh(core_axis_name="c", subcore_axis_name="s")

    @functools.partial(
        pl.kernel,
        out_type=jax.ShapeDtypeStruct((NC, N_pad), jnp.float32),
        mesh=mesh,
        scratch_types=[
            pltpu.VMEM((CHUNK,), jnp.int32),
            pltpu.VMEM((CHUNK,), jnp.float32),
            pltpu.VMEM((rpt,), jnp.float32),
            pltpu.VMEM_SHARED((N_pad,), jnp.float32),
        ],
    )
    def deg_k(dst_hbm, ones_hbm, zeros_hbm, out_hbm, idx_v, ones_v, buf_v, acc_sh):
        c = lax.axis_index("c")
        s = lax.axis_index("s")
        w = c * NS + s
        base = w * per_w
        pltpu.sync_copy(ones_hbm, ones_v)
        pltpu.sync_copy(zeros_hbm.at[pl.ds(0, rpt)], buf_v)
        pltpu.sync_copy(buf_v, acc_sh.at[pl.ds(s * rpt, rpt)])
        plsc.subcore_barrier()

        def body(i, carry):
            off = base + i * CHUNK
            pltpu.sync_copy(dst_hbm.at[pl.ds(off, CHUNK)], idx_v)
            pltpu.sync_copy(ones_v, acc_sh.at[idx_v], add=True)
            return carry

        lax.fori_loop(0, n_chunks, body, 0)
        plsc.subcore_barrier()
        pltpu.sync_copy(acc_sh.at[pl.ds(s * rpt, rpt)], buf_v)
        pltpu.sync_copy(buf_v, out_hbm.at[c].at[pl.ds(s * rpt, rpt)])

    return deg_k


def _make_layer_kernel(N, N_pad, D, E_pad):
    per_w = E_pad // (NC * NS)
    n_chunks = per_w // CHUNK
    rpt = N_pad // NS
    mesh = plsc.VectorSubcoreMesh(core_axis_name="c", subcore_axis_name="s")

    @functools.partial(
        pl.kernel,
        out_type=jax.ShapeDtypeStruct((NC, N_pad, D), jnp.float32),
        mesh=mesh,
        scratch_types=[
            pltpu.VMEM((CHUNK,), jnp.int32),
            pltpu.VMEM((CHUNK,), jnp.int32),
            pltpu.VMEM((CHUNK, D), jnp.float32),
            pltpu.VMEM_SHARED((N_pad, D), jnp.float32),
            pltpu.SemaphoreType.DMA,
        ],
    )
    def layer_k(src_hbm, dst_hbm, g_hbm, zeros_hbm, out_hbm,
                idx_s, idx_d, rows_v, acc_sh, sem):
        c = lax.axis_index("c")
        s = lax.axis_index("s")
        w = c * NS + s
        base = w * per_w
        pltpu.sync_copy(zeros_hbm.at[pl.ds(0, CHUNK)], rows_v)
        n_full0, rem0 = divmod(rpt, CHUNK)
        for j in range(n_full0):
            pltpu.sync_copy(rows_v, acc_sh.at[pl.ds(s * rpt + j * CHUNK, CHUNK)])
        if rem0:
            pltpu.sync_copy(rows_v.at[pl.ds(0, rem0)],
                            acc_sh.at[pl.ds(s * rpt + n_full0 * CHUNK, rem0)])
        plsc.subcore_barrier()

        def body(i, carry):
            off = base + i * CHUNK
            pltpu.sync_copy(src_hbm.at[pl.ds(off, CHUNK)], idx_s)
            pltpu.sync_copy(dst_hbm.at[pl.ds(off, CHUNK)], idx_d)
            pltpu.async_copy(g_hbm.at[idx_s], rows_v, sem).wait()
            pltpu.sync_copy(rows_v, acc_sh.at[idx_d], add=True)
            return carry

        lax.fori_loop(0, n_chunks, body, 0)
        plsc.subcore_barrier()
        n_full, rem = divmod(rpt, CHUNK)
        for j in range(n_full):
            r0 = s * rpt + j * CHUNK
            pltpu.sync_copy(acc_sh.at[pl.ds(r0, CHUNK)], rows_v)
            pltpu.sync_copy(rows_v, out_hbm.at[c].at[pl.ds(r0, CHUNK)])
        if rem:
            r0 = s * rpt + n_full * CHUNK
            pltpu.sync_copy(acc_sh.at[pl.ds(r0, rem)], rows_v.at[pl.ds(0, rem)])
            pltpu.sync_copy(rows_v.at[pl.ds(0, rem)], out_hbm.at[c].at[pl.ds(r0, rem)])

    return layer_k


def _prep1_tc(x, W1, degp2):
    N, D_in = x.shape
    D_h = W1.shape[1]

    def body(x_ref, w_ref, degp_ref, g_ref, dinv_ref):
        dp = degp_ref[...]
        deg = dp[:, 0:1] + dp[:, 1:2] + 1.0
        dinv = lax.rsqrt(deg)
        h = jnp.dot(x_ref[...], w_ref[...], preferred_element_type=jnp.float32)
        g_ref[...] = h * dinv
        dinv_ref[...] = dinv

    return pl.pallas_call(
        body,
        out_shape=[
            jax.ShapeDtypeStruct((N, D_h), jnp.float32),
            jax.ShapeDtypeStruct((N, 1), jnp.float32),
        ],
    )(x, W1, degp2)


def _mid_tc(p, g1, dinv, b1, W2):
    N, D = g1.shape
    D_out = W2.shape[1]

    def body(p_ref, g1_ref, dinv_ref, b1_ref, w2_ref, g2_ref):
        sall = p_ref[0] + p_ref[1] + g1_ref[...]
        z = jnp.maximum(sall * dinv_ref[...] + b1_ref[...], 0.0)
        h2 = jnp.dot(z, w2_ref[...], preferred_element_type=jnp.float32)
        g2_ref[...] = h2 * dinv_ref[...]

    return pl.pallas_call(
        body,
        out_shape=jax.ShapeDtypeStruct((N, D_out), jnp.float32),
    )(p, g1, dinv, b1, W2)


def _final_tc(q, g2, dinv, b2):
    N, D = g2.shape

    def body(q_ref, g2_ref, dinv_ref, b2_ref, o_ref):
        sall = q_ref[0] + q_ref[1] + g2_ref[...]
        o_ref[...] = sall * dinv_ref[...] + b2_ref[...]

    return pl.pallas_call(
        body,
        out_shape=jax.ShapeDtypeStruct((N, D), jnp.float32),
    )(q, g2, dinv, b2)


def kernel(x, edge_index, W1, b1, W2, b2):
    N, D_in = x.shape
    E = edge_index.shape[1]
    D = W1.shape[1]

    W_TOT = NC * NS
    E_pad = ((E + W_TOT * CHUNK - 1) // (W_TOT * CHUNK)) * (W_TOT * CHUNK)
    pad = E_pad - E
    N_pad = ((N + NS * 128) // (NS * 128)) * (NS * 128)
    src = edge_index[0].astype(jnp.int32)
    dst = edge_index[1].astype(jnp.int32)
    if pad:
        ar = jnp.arange(pad, dtype=jnp.int32)
        src = jnp.concatenate([src, ar % N])
        dst = jnp.concatenate([dst, N + ar % (N_pad - N)])

    ones1 = jnp.ones((CHUNK,), jnp.float32)
    zeros1 = jnp.zeros((N_pad,), jnp.float32)
    zerosD = jnp.zeros((N_pad, D), jnp.float32)

    deg_k = _make_deg_kernel(N_pad, E_pad)
    layer_k = _make_layer_kernel(N, N_pad, D, E_pad)

    degp = deg_k(dst, ones1, zeros1)
    degp2 = jnp.transpose(degp[:, :N])

    g1, dinv = _prep1_tc(x, W1, degp2)
    p = layer_k(src, dst, g1, zerosD)
    g2 = _mid_tc(p[:, :N, :], g1, dinv, jnp.reshape(b1, (1, D)), W2)
    q = layer_k(src, dst, g2, zerosD)
    out = _final_tc(q[:, :N, :], g2, dinv, jnp.reshape(b2, (1, W2.shape[1])))
    return out


# speedup vs baseline: 28.1738x; 1.7589x over previous
import functools

import jax
import jax.numpy as jnp
from jax import lax
from jax.experimental import pallas as pl
from jax.experimental.pallas import tpu as pltpu
from jax.experimental.pallas import tpu_sc as plsc

NC = 2
NS = 16
CHUNK = 128


def _make_deg_kernel(N_pad, E_pad):
    per_w = E_pad // (NC * NS)
    n_chunks = per_w // CHUNK
    rpt = N_pad // NS
    mesh = plsc.VectorSubcoreMesh(core_axis_name="c", subcore_axis_name="s")

    @functools.partial(
        pl.kernel,
        out_type=jax.ShapeDtypeStruct((NC, N_pad), jnp.float32),
        mesh=mesh,
        scratch_types=[
            pltpu.VMEM((CHUNK,), jnp.int32),
            pltpu.VMEM((CHUNK,), jnp.float32),
            pltpu.VMEM((rpt,), jnp.float32),
            pltpu.VMEM_SHARED((N_pad,), jnp.float32),
        ],
    )
    def deg_k(dst_hbm, ones_hbm, zeros_hbm, out_hbm, idx_v, ones_v, buf_v, acc_sh):
        c = lax.axis_index("c")
        s = lax.axis_index("s")
        w = c * NS + s
        base = w * per_w
        pltpu.sync_copy(ones_hbm, ones_v)
        pltpu.sync_copy(zeros_hbm.at[pl.ds(0, rpt)], buf_v)
        pltpu.sync_copy(buf_v, acc_sh.at[pl.ds(s * rpt, rpt)])
        plsc.subcore_barrier()

        def body(i, carry):
            off = base + i * CHUNK
            pltpu.sync_copy(dst_hbm.at[pl.ds(off, CHUNK)], idx_v)
            pltpu.sync_copy(ones_v, acc_sh.at[idx_v], add=True)
            return carry

        lax.fori_loop(0, n_chunks, body, 0)
        plsc.subcore_barrier()
        pltpu.sync_copy(acc_sh.at[pl.ds(s * rpt, rpt)], buf_v)
        pltpu.sync_copy(buf_v, out_hbm.at[c].at[pl.ds(s * rpt, rpt)])

    return deg_k


def _make_layer_kernel(N, N_pad, D, E_pad):
    per_w = E_pad // (NC * NS)
    n_chunks = per_w // CHUNK
    assert n_chunks % 2 == 0
    rpt = N_pad // NS
    mesh = plsc.VectorSubcoreMesh(core_axis_name="c", subcore_axis_name="s")

    @functools.partial(
        pl.kernel,
        out_type=jax.ShapeDtypeStruct((NC, N_pad, D), jnp.float32),
        mesh=mesh,
        scratch_types=[
            pltpu.VMEM((n_chunks, CHUNK), jnp.int32),
            pltpu.VMEM((CHUNK,), jnp.int32),
            pltpu.VMEM((CHUNK,), jnp.int32),
            pltpu.VMEM((CHUNK, D), jnp.float32),
            pltpu.VMEM((CHUNK, D), jnp.float32),
            pltpu.VMEM_SHARED((N_pad, D), jnp.float32),
            pltpu.SemaphoreType.DMA,
            pltpu.SemaphoreType.DMA,
            pltpu.SemaphoreType.DMA,
            pltpu.SemaphoreType.DMA,
        ],
    )
    def layer_k(src_hbm, dst_hbm, g_hbm, zeros_hbm, out_hbm,
                src_v, dstb0, dstb1, buf0, buf1, acc_sh,
                semg0, semg1, semd0, semd1):
        c = lax.axis_index("c")
        s = lax.axis_index("s")
        w = c * NS + s
        cbase = w * n_chunks
        pltpu.sync_copy(src_hbm.at[pl.ds(cbase, n_chunks)], src_v)
        pltpu.sync_copy(zeros_hbm, buf0)
        n_full0, rem0 = divmod(rpt, CHUNK)
        for j in range(n_full0):
            pltpu.sync_copy(buf0, acc_sh.at[pl.ds(s * rpt + j * CHUNK, CHUNK)])
        if rem0:
            pltpu.sync_copy(buf0.at[pl.ds(0, rem0)],
                            acc_sh.at[pl.ds(s * rpt + n_full0 * CHUNK, rem0)])
        plsc.subcore_barrier()

        ebase = cbase * CHUNK
        pltpu.async_copy(dst_hbm.at[pl.ds(ebase, CHUNK)], dstb0, semd0)
        pltpu.async_copy(g_hbm.at[src_v.at[0]], buf0, semg0)

        def body(j, carry):
            i0 = 2 * j
            pltpu.async_copy(dst_hbm.at[pl.ds(ebase + (i0 + 1) * CHUNK, CHUNK)],
                             dstb1, semd1)
            pltpu.async_copy(g_hbm.at[src_v.at[i0 + 1]], buf1, semg1)
            pltpu.make_async_copy(g_hbm.at[src_v.at[i0]], buf0, semg0).wait()
            pltpu.make_async_copy(dst_hbm.at[pl.ds(ebase, CHUNK)], dstb0,
                                  semd0).wait()
            pltpu.sync_copy(buf0, acc_sh.at[dstb0], add=True)

            @pl.when(i0 + 2 < n_chunks)
            def _():
                pltpu.async_copy(dst_hbm.at[pl.ds(ebase + (i0 + 2) * CHUNK, CHUNK)],
                                 dstb0, semd0)
                pltpu.async_copy(g_hbm.at[src_v.at[i0 + 2]], buf0, semg0)

            pltpu.make_async_copy(g_hbm.at[src_v.at[i0 + 1]], buf1, semg1).wait()
            pltpu.make_async_copy(dst_hbm.at[pl.ds(ebase, CHUNK)], dstb1,
                                  semd1).wait()
            pltpu.sync_copy(buf1, acc_sh.at[dstb1], add=True)
            return carry

        lax.fori_loop(0, n_chunks // 2, body, 0)
        plsc.subcore_barrier()
        n_full, rem = divmod(rpt, CHUNK)
        for j in range(n_full):
            r0 = s * rpt + j * CHUNK
            pltpu.sync_copy(acc_sh.at[pl.ds(r0, CHUNK)], buf0)
            pltpu.sync_copy(buf0, out_hbm.at[c].at[pl.ds(r0, CHUNK)])
        if rem:
            r0 = s * rpt + n_full * CHUNK
            pltpu.sync_copy(acc_sh.at[pl.ds(r0, rem)], buf0.at[pl.ds(0, rem)])
            pltpu.sync_copy(buf0.at[pl.ds(0, rem)], out_hbm.at[c].at[pl.ds(r0, rem)])

    return layer_k


def _prep1_tc(x, W1, degp2):
    N, D_in = x.shape
    D_h = W1.shape[1]

    def body(x_ref, w_ref, degp_ref, g_ref, dinv_ref):
        dp = degp_ref[...]
        deg = dp[:, 0:1] + dp[:, 1:2] + 1.0
        dinv = lax.rsqrt(deg)
        h = jnp.dot(x_ref[...], w_ref[...], preferred_element_type=jnp.float32)
        g_ref[...] = h * dinv
        dinv_ref[...] = dinv

    return pl.pallas_call(
        body,
        out_shape=[
            jax.ShapeDtypeStruct((N, D_h), jnp.float32),
            jax.ShapeDtypeStruct((N, 1), jnp.float32),
        ],
    )(x, W1, degp2)


def _mid_tc(p, g1, dinv, b1, W2):
    N, D = g1.shape
    D_out = W2.shape[1]

    def body(p_ref, g1_ref, dinv_ref, b1_ref, w2_ref, g2_ref):
        sall = p_ref[0] + p_ref[1] + g1_ref[...]
        z = jnp.maximum(sall * dinv_ref[...] + b1_ref[...], 0.0)
        h2 = jnp.dot(z, w2_ref[...], preferred_element_type=jnp.float32)
        g2_ref[...] = h2 * dinv_ref[...]

    return pl.pallas_call(
        body,
        out_shape=jax.ShapeDtypeStruct((N, D_out), jnp.float32),
    )(p, g1, dinv, b1, W2)


def _final_tc(q, g2, dinv, b2):
    N, D = g2.shape

    def body(q_ref, g2_ref, dinv_ref, b2_ref, o_ref):
        sall = q_ref[0] + q_ref[1] + g2_ref[...]
        o_ref[...] = sall * dinv_ref[...] + b2_ref[...]

    return pl.pallas_call(
        body,
        out_shape=jax.ShapeDtypeStruct((N, D), jnp.float32),
    )(q, g2, dinv, b2)


def kernel(x, edge_index, W1, b1, W2, b2):
    N, D_in = x.shape
    E = edge_index.shape[1]
    D = W1.shape[1]

    W_TOT = NC * NS
    quant = W_TOT * CHUNK * 2
    E_pad = ((E + quant - 1) // quant) * quant
    pad = E_pad - E
    N_pad_l = ((N + 128) // 128) * 128
    N_pad_d = ((N + NS * 128) // (NS * 128)) * (NS * 128)
    src = edge_index[0].astype(jnp.int32)
    dst = edge_index[1].astype(jnp.int32)
    dst_deg = dst
    if pad:
        ar = jnp.arange(pad, dtype=jnp.int32)
        dst_deg = jnp.concatenate([dst, N + ar % (N_pad_d - N)])
        src = jnp.concatenate([src, N + ar % (N_pad_l - N)])
        dst = jnp.concatenate([dst, ar % N_pad_l])

    ones1 = jnp.ones((CHUNK,), jnp.float32)
    zeros1 = jnp.zeros((N_pad_d,), jnp.float32)
    zerosD = jnp.zeros((CHUNK, D), jnp.float32)
    gpad = jnp.zeros((N_pad_l - N, D), jnp.float32)

    deg_k = _make_deg_kernel(N_pad_d, E_pad)
    layer_k = _make_layer_kernel(N, N_pad_l, D, E_pad)

    degp = deg_k(dst_deg, ones1, zeros1)
    degp2 = jnp.transpose(degp[:, :N])

    src2d = jnp.reshape(src, (E_pad // CHUNK, CHUNK))

    g1, dinv = _prep1_tc(x, W1, degp2)
    p = layer_k(src2d, dst, jnp.concatenate([g1, gpad]), zerosD)
    g2 = _mid_tc(p[:, :N, :], g1, dinv, jnp.reshape(b1, (1, D)), W2)
    q = layer_k(src2d, dst, jnp.concatenate([g2, gpad]), zerosD)
    out = _final_tc(q[:, :N, :], g2, dinv, jnp.reshape(b2, (1, W2.shape[1])))
    return out


# speedup vs baseline: 31.4374x; 1.1158x over previous
import functools

import jax
import jax.numpy as jnp
from jax import lax
from jax.experimental import pallas as pl
from jax.experimental.pallas import tpu as pltpu
from jax.experimental.pallas import tpu_sc as plsc

NC = 2
NS = 16
CHUNK = 128


def _make_deg_kernel(N_pad, E_pad):
    per_w = E_pad // (NC * NS)
    n_chunks = per_w // CHUNK
    FIRE = 8 if n_chunks % 8 == 0 else 2
    rpt = N_pad // NS
    mesh = plsc.VectorSubcoreMesh(core_axis_name="c", subcore_axis_name="s")

    @functools.partial(
        pl.kernel,
        out_type=jax.ShapeDtypeStruct((NC, N_pad), jnp.float32),
        mesh=mesh,
        scratch_types=[
            pltpu.VMEM((n_chunks, CHUNK), jnp.int32),
            pltpu.VMEM((CHUNK,), jnp.float32),
            pltpu.VMEM((rpt,), jnp.float32),
            pltpu.VMEM_SHARED((N_pad,), jnp.float32),
            pltpu.SemaphoreType.DMA,
        ],
    )
    def deg_k(dst_hbm, ones_hbm, zeros_hbm, out_hbm, dst_v, ones_v, buf_v,
              acc_sh, sem):
        c = lax.axis_index("c")
        s = lax.axis_index("s")
        w = c * NS + s
        cbase = w * n_chunks
        pltpu.sync_copy(ones_hbm, ones_v)
        pltpu.sync_copy(dst_hbm.at[pl.ds(cbase, n_chunks)], dst_v)
        pltpu.sync_copy(zeros_hbm.at[pl.ds(0, rpt)], buf_v)
        pltpu.sync_copy(buf_v, acc_sh.at[pl.ds(s * rpt, rpt)])
        plsc.subcore_barrier()

        def body(j, carry):
            for b in range(FIRE):
                pltpu.async_copy(ones_v, acc_sh.at[dst_v.at[FIRE * j + b]],
                                 sem, add=True)
            for b in range(FIRE):
                pltpu.make_async_copy(ones_v, acc_sh.at[dst_v.at[0]], sem).wait()
            return carry

        lax.fori_loop(0, n_chunks // FIRE, body, 0)
        plsc.subcore_barrier()
        pltpu.sync_copy(acc_sh.at[pl.ds(s * rpt, rpt)], buf_v)
        pltpu.sync_copy(buf_v, out_hbm.at[c].at[pl.ds(s * rpt, rpt)])

    return deg_k


def _make_layer_kernel(N, N_pad, D, E_pad):
    per_w = E_pad // (NC * NS)
    n_chunks = per_w // CHUNK
    assert n_chunks % 2 == 0
    rpt = N_pad // NS
    mesh = plsc.VectorSubcoreMesh(core_axis_name="c", subcore_axis_name="s")

    @functools.partial(
        pl.kernel,
        out_type=jax.ShapeDtypeStruct((NC, N_pad, D), jnp.float32),
        mesh=mesh,
        scratch_types=[
            pltpu.VMEM((n_chunks, CHUNK), jnp.int32),
            pltpu.VMEM((CHUNK,), jnp.int32),
            pltpu.VMEM((CHUNK,), jnp.int32),
            pltpu.VMEM((CHUNK, D), jnp.float32),
            pltpu.VMEM((CHUNK, D), jnp.float32),
            pltpu.VMEM_SHARED((N_pad, D), jnp.float32),
            pltpu.SemaphoreType.DMA,
            pltpu.SemaphoreType.DMA,
            pltpu.SemaphoreType.DMA,
            pltpu.SemaphoreType.DMA,
        ],
    )
    def layer_k(src_hbm, dst_hbm, g_hbm, zeros_hbm, out_hbm,
                src_v, dstb0, dstb1, buf0, buf1, acc_sh,
                semg0, semg1, semd0, semd1):
        c = lax.axis_index("c")
        s = lax.axis_index("s")
        w = c * NS + s
        cbase = w * n_chunks
        pltpu.sync_copy(src_hbm.at[pl.ds(cbase, n_chunks)], src_v)
        pltpu.sync_copy(zeros_hbm, buf0)
        n_full0, rem0 = divmod(rpt, CHUNK)
        for j in range(n_full0):
            pltpu.sync_copy(buf0, acc_sh.at[pl.ds(s * rpt + j * CHUNK, CHUNK)])
        if rem0:
            pltpu.sync_copy(buf0.at[pl.ds(0, rem0)],
                            acc_sh.at[pl.ds(s * rpt + n_full0 * CHUNK, rem0)])
        plsc.subcore_barrier()

        ebase = cbase * CHUNK
        pltpu.async_copy(dst_hbm.at[pl.ds(ebase, CHUNK)], dstb0, semd0)
        pltpu.async_copy(g_hbm.at[src_v.at[0]], buf0, semg0)

        def body(j, carry):
            i0 = 2 * j
            pltpu.async_copy(dst_hbm.at[pl.ds(ebase + (i0 + 1) * CHUNK, CHUNK)],
                             dstb1, semd1)
            pltpu.async_copy(g_hbm.at[src_v.at[i0 + 1]], buf1, semg1)
            pltpu.make_async_copy(g_hbm.at[src_v.at[i0]], buf0, semg0).wait()
            pltpu.make_async_copy(dst_hbm.at[pl.ds(ebase, CHUNK)], dstb0,
                                  semd0).wait()
            pltpu.sync_copy(buf0, acc_sh.at[dstb0], add=True)

            @pl.when(i0 + 2 < n_chunks)
            def _():
                pltpu.async_copy(dst_hbm.at[pl.ds(ebase + (i0 + 2) * CHUNK, CHUNK)],
                                 dstb0, semd0)
                pltpu.async_copy(g_hbm.at[src_v.at[i0 + 2]], buf0, semg0)

            pltpu.make_async_copy(g_hbm.at[src_v.at[i0 + 1]], buf1, semg1).wait()
            pltpu.make_async_copy(dst_hbm.at[pl.ds(ebase, CHUNK)], dstb1,
                                  semd1).wait()
            pltpu.sync_copy(buf1, acc_sh.at[dstb1], add=True)
            return carry

        lax.fori_loop(0, n_chunks // 2, body, 0)
        plsc.subcore_barrier()
        n_full, rem = divmod(rpt, CHUNK)
        for j in range(n_full):
            r0 = s * rpt + j * CHUNK
            pltpu.sync_copy(acc_sh.at[pl.ds(r0, CHUNK)], buf0)
            pltpu.sync_copy(buf0, out_hbm.at[c].at[pl.ds(r0, CHUNK)])
        if rem:
            r0 = s * rpt + n_full * CHUNK
            pltpu.sync_copy(acc_sh.at[pl.ds(r0, rem)], buf0.at[pl.ds(0, rem)])
            pltpu.sync_copy(buf0.at[pl.ds(0, rem)], out_hbm.at[c].at[pl.ds(r0, rem)])

    return layer_k


def _matmul_tc(x, W1):
    N = x.shape[0]
    D_h = W1.shape[1]

    def body(x_ref, w_ref, h_ref):
        h_ref[...] = jnp.dot(x_ref[...], w_ref[...],
                             preferred_element_type=jnp.float32)

    return pl.pallas_call(
        body, out_shape=jax.ShapeDtypeStruct((N, D_h), jnp.float32),
    )(x, W1)


def _scale_tc(h, degp2):
    N, D_h = h.shape

    def body(h_ref, degp_ref, g_ref, dinv_ref):
        dp = degp_ref[...]
        deg = dp[:, 0:1] + dp[:, 1:2] + 1.0
        dinv = lax.rsqrt(deg)
        g_ref[...] = h_ref[...] * dinv
        dinv_ref[...] = dinv

    return pl.pallas_call(
        body,
        out_shape=[
            jax.ShapeDtypeStruct((N, D_h), jnp.float32),
            jax.ShapeDtypeStruct((N, 1), jnp.float32),
        ],
    )(h, degp2)


def _mid_tc(p, g1, dinv, b1, W2):
    N, D = g1.shape
    D_out = W2.shape[1]

    def body(p_ref, g1_ref, dinv_ref, b1_ref, w2_ref, g2_ref):
        sall = p_ref[0] + p_ref[1] + g1_ref[...]
        z = jnp.maximum(sall * dinv_ref[...] + b1_ref[...], 0.0)
        h2 = jnp.dot(z, w2_ref[...], preferred_element_type=jnp.float32)
        g2_ref[...] = h2 * dinv_ref[...]

    return pl.pallas_call(
        body,
        out_shape=jax.ShapeDtypeStruct((N, D_out), jnp.float32),
    )(p, g1, dinv, b1, W2)


def _final_tc(q, g2, dinv, b2):
    N, D = g2.shape

    def body(q_ref, g2_ref, dinv_ref, b2_ref, o_ref):
        sall = q_ref[0] + q_ref[1] + g2_ref[...]
        o_ref[...] = sall * dinv_ref[...] + b2_ref[...]

    return pl.pallas_call(
        body,
        out_shape=jax.ShapeDtypeStruct((N, D), jnp.float32),
    )(q, g2, dinv, b2)


def kernel(x, edge_index, W1, b1, W2, b2):
    N, D_in = x.shape
    E = edge_index.shape[1]
    D = W1.shape[1]

    W_TOT = NC * NS
    quant = W_TOT * CHUNK * 2
    E_pad = ((E + quant - 1) // quant) * quant
    pad = E_pad - E
    N_pad_l = ((N + 128) // 128) * 128
    N_pad_d = ((N + NS * 128) // (NS * 128)) * (NS * 128)
    src = edge_index[0].astype(jnp.int32)
    dst = edge_index[1].astype(jnp.int32)
    dst_deg = dst
    if pad:
        ar = jnp.arange(pad, dtype=jnp.int32)
        dst_deg = jnp.concatenate([dst, N + ar % (N_pad_d - N)])
        src = jnp.concatenate([src, N + ar % (N_pad_l - N)])
        dst = jnp.concatenate([dst, ar % N_pad_l])

    ones1 = jnp.ones((CHUNK,), jnp.float32)
    zeros1 = jnp.zeros((N_pad_d,), jnp.float32)
    zerosD = jnp.zeros((CHUNK, D), jnp.float32)
    gpad = jnp.zeros((N_pad_l - N, D), jnp.float32)

    deg_k = _make_deg_kernel(N_pad_d, E_pad)
    layer_k = _make_layer_kernel(N, N_pad_l, D, E_pad)

    dst_deg2d = jnp.reshape(dst_deg, (E_pad // CHUNK, CHUNK))
    degp = deg_k(dst_deg2d, ones1, zeros1)
    degp2 = jnp.transpose(degp[:, :N])

    src2d = jnp.reshape(src, (E_pad // CHUNK, CHUNK))

    h1 = _matmul_tc(x, W1)
    g1, dinv = _scale_tc(h1, degp2)
    p = layer_k(src2d, dst, jnp.concatenate([g1, gpad]), zerosD)
    g2 = _mid_tc(p[:, :N, :], g1, dinv, jnp.reshape(b1, (1, D)), W2)
    q = layer_k(src2d, dst, jnp.concatenate([g2, gpad]), zerosD)
    out = _final_tc(q[:, :N, :], g2, dinv, jnp.reshape(b2, (1, W2.shape[1])))
    return out


# speedup vs baseline: 33.9061x; 1.0785x over previous
import functools

import jax
import jax.numpy as jnp
from jax import lax
from jax.experimental import pallas as pl
from jax.experimental.pallas import tpu as pltpu
from jax.experimental.pallas import tpu_sc as plsc

NC = 2
NS = 16
CHUNK = 128


def _make_deg_kernel(N_pad, E_pad):
    per_w = E_pad // (NC * NS)
    n_chunks = per_w // CHUNK
    FIRE = 8 if n_chunks % 8 == 0 else 2
    rpt = N_pad // NS
    mesh = plsc.VectorSubcoreMesh(core_axis_name="c", subcore_axis_name="s")

    @functools.partial(
        pl.kernel,
        out_type=jax.ShapeDtypeStruct((NC, N_pad), jnp.float32),
        mesh=mesh,
        scratch_types=[
            pltpu.VMEM((n_chunks, CHUNK), jnp.int32),
            pltpu.VMEM((CHUNK,), jnp.float32),
            pltpu.VMEM((rpt,), jnp.float32),
            pltpu.VMEM_SHARED((N_pad,), jnp.float32),
            pltpu.SemaphoreType.DMA,
        ],
    )
    def deg_k(dst_hbm, ones_hbm, zeros_hbm, out_hbm, dst_v, ones_v, buf_v,
              acc_sh, sem):
        c = lax.axis_index("c")
        s = lax.axis_index("s")
        w = c * NS + s
        cbase = w * n_chunks
        pltpu.sync_copy(ones_hbm, ones_v)
        pltpu.sync_copy(dst_hbm.at[pl.ds(cbase, n_chunks)], dst_v)
        pltpu.sync_copy(zeros_hbm.at[pl.ds(0, rpt)], buf_v)
        pltpu.sync_copy(buf_v, acc_sh.at[pl.ds(s * rpt, rpt)])
        plsc.subcore_barrier()

        def body(j, carry):
            for b in range(FIRE):
                pltpu.async_copy(ones_v, acc_sh.at[dst_v.at[FIRE * j + b]],
                                 sem, add=True)
            for b in range(FIRE):
                pltpu.make_async_copy(ones_v, acc_sh.at[dst_v.at[0]], sem).wait()
            return carry

        lax.fori_loop(0, n_chunks // FIRE, body, 0)
        plsc.subcore_barrier()
        pltpu.sync_copy(acc_sh.at[pl.ds(s * rpt, rpt)], buf_v)
        pltpu.sync_copy(buf_v, out_hbm.at[c].at[pl.ds(s * rpt, rpt)])

    return deg_k


def _make_layer_kernel(N, N_pad, D, E_pad):
    per_w = E_pad // (NC * NS)
    n_chunks = per_w // CHUNK
    assert n_chunks % 2 == 0
    rpt = N_pad // NS
    mesh = plsc.VectorSubcoreMesh(core_axis_name="c", subcore_axis_name="s")

    @functools.partial(
        pl.kernel,
        out_type=jax.ShapeDtypeStruct((NC, N_pad, D), jnp.float32),
        mesh=mesh,
        scratch_types=[
            pltpu.VMEM((n_chunks, CHUNK), jnp.int32),
            pltpu.VMEM((CHUNK,), jnp.int32),
            pltpu.VMEM((CHUNK,), jnp.int32),
            pltpu.VMEM((CHUNK, D), jnp.float32),
            pltpu.VMEM((CHUNK, D), jnp.float32),
            pltpu.VMEM_SHARED((N_pad, D), jnp.float32),
            pltpu.SemaphoreType.DMA,
            pltpu.SemaphoreType.DMA,
            pltpu.SemaphoreType.DMA,
            pltpu.SemaphoreType.DMA,
        ],
    )
    def layer_k(src_hbm, dst_hbm, g_hbm, zeros_hbm, out_hbm,
                src_v, dstb0, dstb1, buf0, buf1, acc_sh,
                semg0, semg1, semd0, semd1):
        c = lax.axis_index("c")
        s = lax.axis_index("s")
        w = c * NS + s
        cbase = w * n_chunks
        pltpu.sync_copy(src_hbm.at[pl.ds(cbase, n_chunks)], src_v)
        pltpu.sync_copy(zeros_hbm, buf0)
        n_full0, rem0 = divmod(rpt, CHUNK)
        for j in range(n_full0):
            pltpu.sync_copy(buf0, acc_sh.at[pl.ds(s * rpt + j * CHUNK, CHUNK)])
        if rem0:
            pltpu.sync_copy(buf0.at[pl.ds(0, rem0)],
                            acc_sh.at[pl.ds(s * rpt + n_full0 * CHUNK, rem0)])
        plsc.subcore_barrier()

        ebase = cbase * CHUNK
        pltpu.async_copy(dst_hbm.at[pl.ds(ebase, CHUNK)], dstb0, semd0)
        pltpu.async_copy(g_hbm.at[src_v.at[0]], buf0, semg0)

        def body(j, carry):
            i0 = 2 * j
            pltpu.async_copy(dst_hbm.at[pl.ds(ebase + (i0 + 1) * CHUNK, CHUNK)],
                             dstb1, semd1)
            pltpu.async_copy(g_hbm.at[src_v.at[i0 + 1]], buf1, semg1)
            pltpu.make_async_copy(g_hbm.at[src_v.at[i0]], buf0, semg0).wait()
            pltpu.make_async_copy(dst_hbm.at[pl.ds(ebase, CHUNK)], dstb0,
                                  semd0).wait()
            pltpu.sync_copy(buf0, acc_sh.at[dstb0], add=True)

            @pl.when(i0 + 2 < n_chunks)
            def _():
                pltpu.async_copy(dst_hbm.at[pl.ds(ebase + (i0 + 2) * CHUNK, CHUNK)],
                                 dstb0, semd0)
                pltpu.async_copy(g_hbm.at[src_v.at[i0 + 2]], buf0, semg0)

            pltpu.make_async_copy(g_hbm.at[src_v.at[i0 + 1]], buf1, semg1).wait()
            pltpu.make_async_copy(dst_hbm.at[pl.ds(ebase, CHUNK)], dstb1,
                                  semd1).wait()
            pltpu.sync_copy(buf1, acc_sh.at[dstb1], add=True)
            return carry

        lax.fori_loop(0, n_chunks // 2, body, 0)
        plsc.subcore_barrier()
        n_full, rem = divmod(rpt, CHUNK)
        for j in range(n_full):
            r0 = s * rpt + j * CHUNK
            pltpu.sync_copy(acc_sh.at[pl.ds(r0, CHUNK)], buf0)
            pltpu.sync_copy(buf0, out_hbm.at[c].at[pl.ds(r0, CHUNK)])
        if rem:
            r0 = s * rpt + n_full * CHUNK
            pltpu.sync_copy(acc_sh.at[pl.ds(r0, rem)], buf0.at[pl.ds(0, rem)])
            pltpu.sync_copy(buf0.at[pl.ds(0, rem)], out_hbm.at[c].at[pl.ds(r0, rem)])

    return layer_k


def _matmul_tc(x, W1, N_pad):
    N = x.shape[0]
    D_h = W1.shape[1]

    def body(x_ref, w_ref, h_ref):
        h_ref[pl.ds(0, N), :] = jnp.dot(x_ref[...], w_ref[...],
                                        preferred_element_type=jnp.float32)
        h_ref[pl.ds(N, N_pad - N), :] = jnp.zeros((N_pad - N, D_h), jnp.float32)

    return pl.pallas_call(
        body, out_shape=jax.ShapeDtypeStruct((N_pad, D_h), jnp.float32),
    )(x, W1)


def _scale_tc(h, degp2):
    N_pad, D_h = h.shape

    def body(h_ref, degp_ref, g_ref, dinv_ref):
        dp = degp_ref[...]
        deg = dp[:, 0:1] + dp[:, 1:2] + 1.0
        dinv = lax.rsqrt(deg)
        g_ref[...] = h_ref[...] * dinv
        dinv_ref[...] = dinv

    return pl.pallas_call(
        body,
        out_shape=[
            jax.ShapeDtypeStruct((N_pad, D_h), jnp.float32),
            jax.ShapeDtypeStruct((N_pad, 1), jnp.float32),
        ],
    )(h, degp2)


def _mid_tc(p, g1, dinv, b1, W2, N):
    N_pad, D = g1.shape
    D_out = W2.shape[1]

    def body(p_ref, g1_ref, dinv_ref, b1_ref, w2_ref, g2_ref):
        sall = p_ref[0] + p_ref[1] + g1_ref[...]
        z = jnp.maximum(sall * dinv_ref[...] + b1_ref[...], 0.0)
        h2 = jnp.dot(z, w2_ref[...], preferred_element_type=jnp.float32)
        g2 = h2 * dinv_ref[...]
        g2_ref[pl.ds(0, N), :] = g2[:N]
        g2_ref[pl.ds(N, N_pad - N), :] = jnp.zeros((N_pad - N, D_out),
                                                   jnp.float32)

    return pl.pallas_call(
        body,
        out_shape=jax.ShapeDtypeStruct((N_pad, D_out), jnp.float32),
    )(p, g1, dinv, b1, W2)


def _final_tc(q, g2, dinv, b2, N):
    N_pad, D = g2.shape

    def body(q_ref, g2_ref, dinv_ref, b2_ref, o_ref):
        sall = q_ref[0, pl.ds(0, N), :] + q_ref[1, pl.ds(0, N), :] + \
            g2_ref[pl.ds(0, N), :]
        o_ref[...] = sall * dinv_ref[pl.ds(0, N), :] + b2_ref[...]

    return pl.pallas_call(
        body,
        out_shape=jax.ShapeDtypeStruct((N, D), jnp.float32),
    )(q, g2, dinv, b2)


def kernel(x, edge_index, W1, b1, W2, b2):
    N, D_in = x.shape
    E = edge_index.shape[1]
    D = W1.shape[1]

    W_TOT = NC * NS
    quant = W_TOT * CHUNK * 2
    E_pad = ((E + quant - 1) // quant) * quant
    pad = E_pad - E
    N_pad_l = ((N + 128) // 128) * 128
    N_pad_d = ((N + NS * 128) // (NS * 128)) * (NS * 128)
    src = edge_index[0].astype(jnp.int32)
    dst = edge_index[1].astype(jnp.int32)
    dst_deg = dst
    if pad:
        ar = jnp.arange(pad, dtype=jnp.int32)
        dst_deg = jnp.concatenate([dst, N + ar % (N_pad_d - N)])
        src = jnp.concatenate([src, N + ar % (N_pad_l - N)])
        dst = jnp.concatenate([dst, ar % N_pad_l])

    ones1 = jnp.ones((CHUNK,), jnp.float32)
    zeros1 = jnp.zeros((N_pad_d,), jnp.float32)
    zerosD = jnp.zeros((CHUNK, D), jnp.float32)

    deg_k = _make_deg_kernel(N_pad_d, E_pad)
    layer_k = _make_layer_kernel(N, N_pad_l, D, E_pad)

    dst_deg2d = jnp.reshape(dst_deg, (E_pad // CHUNK, CHUNK))
    degp = deg_k(dst_deg2d, ones1, zeros1)
    degp2 = jnp.transpose(degp[:, :N_pad_l])

    src2d = jnp.reshape(src, (E_pad // CHUNK, CHUNK))

    h1 = _matmul_tc(x, W1, N_pad_l)
    g1, dinv = _scale_tc(h1, degp2)
    p = layer_k(src2d, dst, g1, zerosD)
    g2 = _mid_tc(p, g1, dinv, jnp.reshape(b1, (1, D)), W2, N)
    q = layer_k(src2d, dst, g2, zerosD)
    out = _final_tc(q, g2, dinv, jnp.reshape(b2, (1, W2.shape[1])), N)
    return out


# speedup vs baseline: 34.3584x; 1.0133x over previous
import functools

import jax
import jax.numpy as jnp
from jax import lax
from jax.experimental import pallas as pl
from jax.experimental.pallas import tpu as pltpu
from jax.experimental.pallas import tpu_sc as plsc

NC = 2
NS = 16
CHUNK = 128


def _make_deg_kernel(N_pad, E_pad):
    per_w = E_pad // (NC * NS)
    n_chunks = per_w // CHUNK
    FIRE = 8 if n_chunks % 8 == 0 else 2
    rpt = N_pad // NS
    mesh = plsc.VectorSubcoreMesh(core_axis_name="c", subcore_axis_name="s")

    @functools.partial(
        pl.kernel,
        out_type=jax.ShapeDtypeStruct((NC, N_pad), jnp.float32),
        mesh=mesh,
        scratch_types=[
            pltpu.VMEM((n_chunks, CHUNK), jnp.int32),
            pltpu.VMEM((CHUNK,), jnp.float32),
            pltpu.VMEM((rpt,), jnp.float32),
            pltpu.VMEM_SHARED((N_pad,), jnp.float32),
            pltpu.SemaphoreType.DMA,
        ],
    )
    def deg_k(dst_hbm, ones_hbm, zeros_hbm, out_hbm, dst_v, ones_v, buf_v,
              acc_sh, sem):
        c = lax.axis_index("c")
        s = lax.axis_index("s")
        w = c * NS + s
        cbase = w * n_chunks
        pltpu.sync_copy(ones_hbm, ones_v)
        pltpu.sync_copy(dst_hbm.at[pl.ds(cbase, n_chunks)], dst_v)
        pltpu.sync_copy(zeros_hbm.at[pl.ds(0, rpt)], buf_v)
        pltpu.sync_copy(buf_v, acc_sh.at[pl.ds(s * rpt, rpt)])
        plsc.subcore_barrier()

        def body(j, carry):
            for b in range(FIRE):
                pltpu.async_copy(ones_v, acc_sh.at[dst_v.at[FIRE * j + b]],
                                 sem, add=True)
            for b in range(FIRE):
                pltpu.make_async_copy(ones_v, acc_sh.at[dst_v.at[0]], sem).wait()
            return carry

        lax.fori_loop(0, n_chunks // FIRE, body, 0)
        plsc.subcore_barrier()
        pltpu.sync_copy(acc_sh.at[pl.ds(s * rpt, rpt)], buf_v)
        pltpu.sync_copy(buf_v, out_hbm.at[c].at[pl.ds(s * rpt, rpt)])

    return deg_k


def _make_layer_kernel(N, N_pad, D, E_pad):
    per_w = E_pad // (NC * NS)
    n_chunks = per_w // CHUNK
    assert n_chunks % 2 == 0
    rpt = N_pad // NS
    mesh = plsc.VectorSubcoreMesh(core_axis_name="c", subcore_axis_name="s")

    @functools.partial(
        pl.kernel,
        out_type=jax.ShapeDtypeStruct((NC, N_pad, D), jnp.float32),
        mesh=mesh,
        scratch_types=[
            pltpu.VMEM((n_chunks, CHUNK), jnp.int32),
            pltpu.VMEM((CHUNK,), jnp.int32),
            pltpu.VMEM((CHUNK,), jnp.int32),
            pltpu.VMEM((CHUNK, D), jnp.float32),
            pltpu.VMEM((CHUNK, D), jnp.float32),
            pltpu.VMEM_SHARED((N_pad, D), jnp.float32),
            pltpu.SemaphoreType.DMA,
            pltpu.SemaphoreType.DMA,
            pltpu.SemaphoreType.DMA,
            pltpu.SemaphoreType.DMA,
        ],
    )
    def layer_k(src_hbm, dst_hbm, g_hbm, zeros_hbm, out_hbm,
                src_v, dstb0, dstb1, buf0, buf1, acc_sh,
                semg0, semg1, semd0, semd1):
        c = lax.axis_index("c")
        s = lax.axis_index("s")
        w = c * NS + s
        cbase = w * n_chunks
        pltpu.sync_copy(src_hbm.at[pl.ds(cbase, n_chunks)], src_v)
        pltpu.sync_copy(zeros_hbm, buf0)
        n_full0, rem0 = divmod(rpt, CHUNK)
        for j in range(n_full0):
            pltpu.sync_copy(buf0, acc_sh.at[pl.ds(s * rpt + j * CHUNK, CHUNK)])
        if rem0:
            pltpu.sync_copy(buf0.at[pl.ds(0, rem0)],
                            acc_sh.at[pl.ds(s * rpt + n_full0 * CHUNK, rem0)])
        plsc.subcore_barrier()

        ebase = cbase * CHUNK
        pltpu.async_copy(dst_hbm.at[pl.ds(ebase, CHUNK)], dstb0, semd0)
        pltpu.async_copy(g_hbm.at[src_v.at[0]], buf0, semg0)

        def body(j, carry):
            i0 = 2 * j
            pltpu.async_copy(dst_hbm.at[pl.ds(ebase + (i0 + 1) * CHUNK, CHUNK)],
                             dstb1, semd1)
            pltpu.async_copy(g_hbm.at[src_v.at[i0 + 1]], buf1, semg1)
            pltpu.make_async_copy(g_hbm.at[src_v.at[i0]], buf0, semg0).wait()
            pltpu.make_async_copy(dst_hbm.at[pl.ds(ebase, CHUNK)], dstb0,
                                  semd0).wait()
            pltpu.sync_copy(buf0, acc_sh.at[dstb0], add=True)

            @pl.when(i0 + 2 < n_chunks)
            def _():
                pltpu.async_copy(dst_hbm.at[pl.ds(ebase + (i0 + 2) * CHUNK, CHUNK)],
                                 dstb0, semd0)
                pltpu.async_copy(g_hbm.at[src_v.at[i0 + 2]], buf0, semg0)

            pltpu.make_async_copy(g_hbm.at[src_v.at[i0 + 1]], buf1, semg1).wait()
            pltpu.make_async_copy(dst_hbm.at[pl.ds(ebase, CHUNK)], dstb1,
                                  semd1).wait()
            pltpu.sync_copy(buf1, acc_sh.at[dstb1], add=True)
            return carry

        lax.fori_loop(0, n_chunks // 2, body, 0)
        plsc.subcore_barrier()
        n_full, rem = divmod(rpt, CHUNK)
        for j in range(n_full):
            r0 = s * rpt + j * CHUNK
            pltpu.sync_copy(acc_sh.at[pl.ds(r0, CHUNK)], buf0)
            pltpu.sync_copy(buf0, out_hbm.at[c].at[pl.ds(r0, CHUNK)])
        if rem:
            r0 = s * rpt + n_full * CHUNK
            pltpu.sync_copy(acc_sh.at[pl.ds(r0, rem)], buf0.at[pl.ds(0, rem)])
            pltpu.sync_copy(buf0.at[pl.ds(0, rem)], out_hbm.at[c].at[pl.ds(r0, rem)])

    return layer_k


def _prep_tc(x, W1, degp2, N_pad):
    N = x.shape[0]
    D_h = W1.shape[1]

    def body(x_ref, w_ref, degp_ref, g_ref, dinv_ref):
        dp = degp_ref[...]
        deg = dp[:, 0:1] + dp[:, 1:2] + 1.0
        dinv = lax.rsqrt(deg)
        h = jnp.dot(x_ref[...], w_ref[...], preferred_element_type=jnp.float32)
        g_ref[pl.ds(0, N), :] = h * dinv[:N]
        g_ref[pl.ds(N, N_pad - N), :] = jnp.zeros((N_pad - N, D_h), jnp.float32)
        dinv_ref[...] = dinv

    return pl.pallas_call(
        body,
        out_shape=[
            jax.ShapeDtypeStruct((N_pad, D_h), jnp.float32),
            jax.ShapeDtypeStruct((N_pad, 1), jnp.float32),
        ],
    )(x, W1, degp2)


def _matmul_tc(x, W1, N_pad):
    N = x.shape[0]
    D_h = W1.shape[1]

    def body(x_ref, w_ref, h_ref):
        h_ref[pl.ds(0, N), :] = jnp.dot(x_ref[...], w_ref[...],
                                        preferred_element_type=jnp.float32)
        h_ref[pl.ds(N, N_pad - N), :] = jnp.zeros((N_pad - N, D_h), jnp.float32)

    return pl.pallas_call(
        body, out_shape=jax.ShapeDtypeStruct((N_pad, D_h), jnp.float32),
    )(x, W1)


def _scale_tc(h, degp2):
    N_pad, D_h = h.shape

    def body(h_ref, degp_ref, g_ref, dinv_ref):
        dp = degp_ref[...]
        deg = dp[:, 0:1] + dp[:, 1:2] + 1.0
        dinv = lax.rsqrt(deg)
        g_ref[...] = h_ref[...] * dinv
        dinv_ref[...] = dinv

    return pl.pallas_call(
        body,
        out_shape=[
            jax.ShapeDtypeStruct((N_pad, D_h), jnp.float32),
            jax.ShapeDtypeStruct((N_pad, 1), jnp.float32),
        ],
    )(h, degp2)


def _mid_tc(p, g1, dinv, b1, W2, N):
    N_pad, D = g1.shape
    D_out = W2.shape[1]

    def body(p_ref, g1_ref, dinv_ref, b1_ref, w2_ref, g2_ref):
        sall = p_ref[0] + p_ref[1] + g1_ref[...]
        z = jnp.maximum(sall * dinv_ref[...] + b1_ref[...], 0.0)
        h2 = jnp.dot(z, w2_ref[...], preferred_element_type=jnp.float32)
        g2 = h2 * dinv_ref[...]
        g2_ref[pl.ds(0, N), :] = g2[:N]
        g2_ref[pl.ds(N, N_pad - N), :] = jnp.zeros((N_pad - N, D_out),
                                                   jnp.float32)

    return pl.pallas_call(
        body,
        out_shape=jax.ShapeDtypeStruct((N_pad, D_out), jnp.float32),
    )(p, g1, dinv, b1, W2)


def _final_tc(q, g2, dinv, b2, N):
    N_pad, D = g2.shape

    def body(q_ref, g2_ref, dinv_ref, b2_ref, o_ref):
        sall = q_ref[0, pl.ds(0, N), :] + q_ref[1, pl.ds(0, N), :] + \
            g2_ref[pl.ds(0, N), :]
        o_ref[...] = sall * dinv_ref[pl.ds(0, N), :] + b2_ref[...]

    return pl.pallas_call(
        body,
        out_shape=jax.ShapeDtypeStruct((N, D), jnp.float32),
    )(q, g2, dinv, b2)


def kernel(x, edge_index, W1, b1, W2, b2):
    N, D_in = x.shape
    E = edge_index.shape[1]
    D = W1.shape[1]

    W_TOT = NC * NS
    quant = W_TOT * CHUNK * 2
    E_pad = ((E + quant - 1) // quant) * quant
    pad = E_pad - E
    N_pad_l = ((N + 128) // 128) * 128
    N_pad_d = ((N + NS * 128) // (NS * 128)) * (NS * 128)
    src = edge_index[0].astype(jnp.int32)
    dst = edge_index[1].astype(jnp.int32)
    dst_deg = dst
    if pad:
        ar = jnp.arange(pad, dtype=jnp.int32)
        dst_deg = jnp.concatenate([dst, N + ar % (N_pad_d - N)])
        src = jnp.concatenate([src, N + ar % (N_pad_l - N)])
        dst = jnp.concatenate([dst, ar % N_pad_l])

    ones1 = jnp.ones((CHUNK,), jnp.float32)
    zeros1 = jnp.zeros((N_pad_d,), jnp.float32)
    zerosD = jnp.zeros((CHUNK, D), jnp.float32)

    deg_k = _make_deg_kernel(N_pad_d, E_pad)
    layer_k = _make_layer_kernel(N, N_pad_l, D, E_pad)

    dst_deg2d = jnp.reshape(dst_deg, (E_pad // CHUNK, CHUNK))
    degp = deg_k(dst_deg2d, ones1, zeros1)
    degp2 = jnp.transpose(degp[:, :N_pad_l])

    src2d = jnp.reshape(src, (E_pad // CHUNK, CHUNK))

    g1, dinv = _prep_tc(x, W1, degp2, N_pad_l)
    p = layer_k(src2d, dst, g1, zerosD)
    g2 = _mid_tc(p, g1, dinv, jnp.reshape(b1, (1, D)), W2, N)
    q = layer_k(src2d, dst, g2, zerosD)
    out = _final_tc(q, g2, dinv, jnp.reshape(b2, (1, W2.shape[1])), N)
    return out
